# Initial kernel scaffold; baseline (speedup 1.0000x reference)
import jax
import jax.numpy as jnp
from jax.experimental import pallas as pl


def kernel(idx_flat, latents, idxcache, relpos_cache):
    raise NotImplementedError("write your pallas kernel here")


# speedup vs baseline: 1.3822x; 1.3822x over previous
import functools

import jax
import jax.numpy as jnp
from jax import lax
from jax.experimental import pallas as pl
from jax.experimental.pallas import tpu as pltpu
from jax.experimental.pallas import tpu_sc as plsc

NEI = 8
D = 64
REL = NEI * 2
LAT = NEI * D
SUB = LAT // 16
ROWS = SUB + 1
L = 16


def kernel(idx_flat, latents, idxcache, relpos_cache):
    B = idx_flat.shape[0]
    grid = idxcache.shape[0]
    lat16 = latents.reshape(latents.shape[0] * (D // 16), 16)
    cache_flat = idxcache.reshape(grid * NEI)
    relpos16 = relpos_cache.reshape(grid, REL)

    info = plsc.get_sparse_core_info()
    nw = info.num_cores * info.num_subcores
    per_w = B // nw
    C = 128
    n_chunks = per_w // C

    mesh = plsc.VectorSubcoreMesh(core_axis_name="c", subcore_axis_name="s")

    @functools.partial(
        pl.kernel,
        mesh=mesh,
        compiler_params=pltpu.CompilerParams(
            needs_layout_passes=False, use_tc_tiling_on_sc=False),
        out_type=jax.ShapeDtypeStruct((B * ROWS, 16), jnp.float32),
        scratch_types=[
            pltpu.VMEM((per_w,), jnp.int32),
            pltpu.VMEM((C * NEI,), jnp.int32),
            pltpu.VMEM((C * NEI,), jnp.int32),
            pltpu.VMEM((C * SUB,), jnp.int32),
            pltpu.VMEM((C * SUB, 16), jnp.float32),
            pltpu.VMEM((C, REL), jnp.float32),
            pltpu.VMEM((C * SUB,), jnp.int32),
            pltpu.VMEM((C,), jnp.int32),
            pltpu.SemaphoreType.DMA,
        ],
    )
    def run(idx_hbm, lat_hbm, cache_hbm, rel_hbm, out_hbm,
            idx_v, fidx_v, nbr_v, lidx_v, lat_v, rel_v, oidx_v, ridx_v, sem):
        wid = lax.axis_index("s") * info.num_cores + lax.axis_index("c")
        base = wid * per_w
        pltpu.sync_copy(idx_hbm.at[pl.ds(base, per_w)], idx_v)
        iota = lax.iota(jnp.int32, L)
        for ci in range(n_chunks):
            row0 = base + ci * C
            for k in range(C // L):
                g8 = idx_v[pl.ds(ci * C + k * L, L)] * NEI
                pos = iota * NEI + k * L * NEI
                for j in range(NEI):
                    plsc.store_scatter(fidx_v, [pos + j], g8 + j)
            pltpu.async_copy(cache_hbm.at[fidx_v], nbr_v, sem).wait()
            for k in range(C * NEI // L):
                n4 = nbr_v[pl.ds(k * L, L)] * 4
                pos = iota * 4 + k * L * 4
                for j in range(4):
                    plsc.store_scatter(lidx_v, [pos + j], n4 + j)
            for k in range(C * SUB // L):
                e = k // 2
                s0 = (k % 2) * L
                oidx_v[pl.ds(k * L, L)] = iota + ((row0 + e) * ROWS + s0)
            for k in range(C // L):
                ridx_v[pl.ds(k * L, L)] = (iota + (row0 + k * L)) * ROWS + SUB
            lat_cp = pltpu.async_copy(lat_hbm.at[lidx_v], lat_v, sem)
            rel_cp = pltpu.async_copy(
                rel_hbm.at[idx_v.at[pl.ds(ci * C, C)]], rel_v, sem)
            lat_cp.wait()
            rel_cp.wait()
            lat_st = pltpu.async_copy(lat_v, out_hbm.at[oidx_v], sem)
            rel_st = pltpu.async_copy(rel_v, out_hbm.at[ridx_v], sem)
            lat_st.wait()
            rel_st.wait()

    out = run(idx_flat, lat16, cache_flat, relpos16)
    return out.reshape(B, ROWS * 16)


# speedup vs baseline: 2.6020x; 1.8826x over previous
import functools

import jax
import jax.numpy as jnp
from jax import lax
from jax.experimental import pallas as pl
from jax.experimental.pallas import tpu as pltpu
from jax.experimental.pallas import tpu_sc as plsc

NEI = 8
D = 64
REL = NEI * 2
LAT = NEI * D
SUB = LAT // 16
ROWS = SUB + 1
L = 16


def kernel(idx_flat, latents, idxcache, relpos_cache):
    B = idx_flat.shape[0]
    grid = idxcache.shape[0]
    gtiles = grid // 128
    lat16 = latents.reshape(latents.shape[0] * (D // 16), 16)
    cache_nat = idxcache.reshape(gtiles, 128, NEI).transpose(0, 2, 1).reshape(grid * NEI)
    rel_nat = relpos_cache.reshape(gtiles, 128, NEI, 2).transpose(2, 0, 3, 1).reshape(grid * REL)

    info = plsc.get_sparse_core_info()
    nw = info.num_cores * info.num_subcores
    per_w = B // nw
    C = 128
    n_chunks = per_w // C

    mesh = plsc.VectorSubcoreMesh(core_axis_name="c", subcore_axis_name="s")

    @functools.partial(
        pl.kernel,
        mesh=mesh,
        compiler_params=pltpu.CompilerParams(
            needs_layout_passes=False, use_tc_tiling_on_sc=False),
        out_type=jax.ShapeDtypeStruct((B * ROWS, 16), jnp.float32),
        scratch_types=[
            pltpu.VMEM((per_w,), jnp.int32),
            pltpu.VMEM((C * NEI,), jnp.int32),
            pltpu.VMEM((C * NEI,), jnp.int32),
            pltpu.VMEM((C * SUB,), jnp.int32),
            pltpu.VMEM((C * SUB, 16), jnp.float32),
            pltpu.VMEM((C * REL,), jnp.int32),
            pltpu.VMEM((C * REL,), jnp.float32),
            pltpu.VMEM((C, REL), jnp.float32),
            pltpu.VMEM((C * SUB,), jnp.int32),
            pltpu.VMEM((C,), jnp.int32),
            pltpu.SemaphoreType.DMA,
        ],
    )
    def run(idx_hbm, lat_hbm, cache_hbm, rel_hbm, out_hbm,
            idx_v, fidx_v, nbr_v, lidx_v, lat_v, ridx16_v, rel1_v, rel_v,
            oidx_v, ridx_v, sem):
        wid = lax.axis_index("s") * info.num_cores + lax.axis_index("c")
        base = wid * per_w
        pltpu.sync_copy(idx_hbm.at[pl.ds(base, per_w)], idx_v)
        iota = lax.iota(jnp.int32, L)

        @pl.loop(0, n_chunks)
        def _chunk(ci):
            row0 = base + ci * C
            for k in range(C // L):
                g = idx_v[pl.ds(ci * C + k * L, L)]
                t = lax.shift_right_logical(g, 7)
                m = lax.bitwise_and(g, 127)
                cbase = lax.shift_left(t, 10) + m
                pos = iota * NEI + k * L * NEI
                for j in range(NEI):
                    plsc.store_scatter(fidx_v, [pos + j], cbase + j * 128)
                rbase = lax.shift_left(t, 8) + m
                rpos = iota * REL + k * L * REL
                for j in range(NEI):
                    for k2 in range(2):
                        plsc.store_scatter(
                            ridx16_v, [rpos + (j * 2 + k2)],
                            rbase + (j * grid * 2 + k2 * 128))
            cache_cp = pltpu.async_copy(cache_hbm.at[fidx_v], nbr_v, sem)
            rel_cp = pltpu.async_copy(rel_hbm.at[ridx16_v], rel1_v, sem)
            cache_cp.wait()
            for k in range(C * NEI // L):
                n4 = nbr_v[pl.ds(k * L, L)] * 4
                pos = iota * 4 + k * L * 4
                for j in range(4):
                    plsc.store_scatter(lidx_v, [pos + j], n4 + j)
            lat_cp = pltpu.async_copy(lat_hbm.at[lidx_v], lat_v, sem)
            for k in range(C * SUB // L):
                e = k // 2
                s0 = (k % 2) * L
                oidx_v[pl.ds(k * L, L)] = iota + ((row0 + e) * ROWS + s0)
            for k in range(C // L):
                ridx_v[pl.ds(k * L, L)] = (iota + (row0 + k * L)) * ROWS + SUB
            rel_cp.wait()
            for e in range(C):
                rel_v[e, :] = rel1_v[pl.ds(e * REL, REL)]
            lat_cp.wait()
            lat_st = pltpu.async_copy(lat_v, out_hbm.at[oidx_v], sem)
            rel_st = pltpu.async_copy(rel_v, out_hbm.at[ridx_v], sem)
            lat_st.wait()
            rel_st.wait()

    out = run(idx_flat, lat16, cache_nat, rel_nat)
    return out.reshape(B, ROWS * 16)
